# baseline (speedup 1.0000x reference)
import functools

import jax
import jax.numpy as jnp
from jax import lax
from jax.experimental import pallas as pl
from jax.experimental.pallas import tpu as pltpu
from jax.experimental.pallas import tpu_sc as plsc


@functools.partial(jax.jit, static_argnums=(0, 1, 2))
def _embed(nb, t, d_half, idx_flat, tok_table, pos_table):
    NW = 32
    L = 16
    assert nb == NW * 128
    assert d_half == 32
    rpw = 128 * t

    mesh = plsc.VectorSubcoreMesh(core_axis_name="c", subcore_axis_name="s")

    @functools.partial(
        pl.kernel,
        out_type=jax.ShapeDtypeStruct((t, 8, NW, 8, 128), jnp.float32),
        mesh=mesh,
        scratch_types=[
            pltpu.VMEM((rpw,), jnp.int32),
            pltpu.VMEM((t, d_half), jnp.float32),
            pltpu.VMEM((2, 128), jnp.int32),
            pltpu.VMEM((2, 128, d_half), jnp.float32),
            pltpu.VMEM((2, 8, 8, 128), jnp.float32),
            pltpu.SemaphoreType.DMA,
            pltpu.SemaphoreType.DMA,
            pltpu.SemaphoreType.DMA,
            pltpu.SemaphoreType.DMA,
            pltpu.SemaphoreType.DMA,
        ],
        compiler_params=pltpu.CompilerParams(use_tc_tiling_on_sc=False,
                                             needs_layout_passes=False),
    )
    def emb(idx_hbm, tok_hbm, pos_hbm, out_hbm,
            idxb_v, pos_v, gidx_v, g_v, s_v,
            sem_g0, sem_g1, sem_w0, sem_w1, sem_x):
        sem_g = (sem_g0, sem_g1)
        sem_w = (sem_w0, sem_w1)
        wid = lax.axis_index("s") * 2 + lax.axis_index("c")

        pltpu.async_copy(pos_hbm.at[pl.ds(0, t)], pos_v, sem_x)
        pltpu.sync_copy(idx_hbm.at[pl.ds(wid * rpw, rpw)], idxb_v)
        pltpu.make_async_copy(pos_hbm.at[pl.ds(0, t)], pos_v, sem_x).wait()

        iota = jnp.arange(L, dtype=jnp.int32)
        iota_t = iota * t

        def extract_gidx(tt, p):
            for j in range(8):
                addr = iota_t + (j * L * t + tt)
                gidx_v[p, pl.ds(j * L, L)] = plsc.load_gather(idxb_v, [addr])

        def fire_gather(tt, p):
            extract_gidx(tt, p)
            pltpu.async_copy(tok_hbm.at[gidx_v.at[p]], g_v.at[p], sem_g[p])

        def sub(tt, p):
            @pl.when(tt >= 2)
            def _():
                pltpu.make_async_copy(
                    s_v.at[p], out_hbm.at[tt, :, wid], sem_w[p]).wait()

            pltpu.make_async_copy(
                tok_hbm.at[gidx_v.at[p]], g_v.at[p], sem_g[p]).wait()

            @pl.when(tt + 1 < t)
            def _():
                fire_gather(tt + 1, 1 - p)

            for f in range(d_half):
                col = jnp.full((L,), f, jnp.int32)
                for j in range(8):
                    row = iota + (j * L)
                    s_v[p, f // 8, f % 8, pl.ds(j * L, L)] = (
                        plsc.load_gather(g_v.at[p], [row, col]))

            for f in range(d_half):
                pv = plsc.load_gather(
                    pos_v, [jnp.full((L,), tt, jnp.int32),
                            jnp.full((L,), f, jnp.int32)])
                for j in range(8):
                    s_v[p, 4 + f // 8, f % 8, pl.ds(j * L, L)] = pv

            pltpu.async_copy(s_v.at[p], out_hbm.at[tt, :, wid], sem_w[p])

        fire_gather(0, 0)

        @pl.loop(0, t, step=2)
        def body(g):
            sub(g, 0)
            sub(g + 1, 1)

        for p in range(2):
            pltpu.make_async_copy(
                s_v.at[p], out_hbm.at[0, :, wid], sem_w[p]).wait()

    return emb(idx_flat, tok_table, pos_table)


def kernel(idx, tok_table, pos_table):
    B, T = idx.shape
    d_half = tok_table.shape[1]
    out5 = _embed(B, T, d_half, idx.reshape(-1).astype(jnp.int32),
                  tok_table, pos_table)
    return out5.transpose(2, 4, 0, 1, 3).reshape(B, T, 2 * d_half)


# speedup vs baseline: 1.4893x; 1.4893x over previous
import functools

import jax
import jax.numpy as jnp
from jax import lax
from jax.experimental import pallas as pl
from jax.experimental.pallas import tpu as pltpu
from jax.experimental.pallas import tpu_sc as plsc


@functools.partial(jax.jit, static_argnums=(0, 1, 2))
def _embed(nb, t, d_half, idx_flat, tok_table, pos_table):
    NW = 32
    L = 16
    assert nb == NW * 128
    assert d_half == 32
    rpw = 128 * t

    mesh = plsc.VectorSubcoreMesh(core_axis_name="c", subcore_axis_name="s")

    @functools.partial(
        pl.kernel,
        out_type=jax.ShapeDtypeStruct((t, 8 * NW * 8 * 128), jnp.float32),
        mesh=mesh,
        scratch_types=[
            pltpu.VMEM((rpw,), jnp.int32),
            pltpu.VMEM((t, d_half), jnp.float32),
            pltpu.VMEM((2, 128), jnp.int32),
            pltpu.VMEM((2, 128, 32), jnp.float32),
            pltpu.VMEM((2, 4 * 8 * 128), jnp.float32),
            pltpu.VMEM((4 * 8 * 128,), jnp.float32),
            pltpu.VMEM_SHARED((t, 4 * 8 * 128), jnp.float32),
            pltpu.SemaphoreType.DMA,
            pltpu.SemaphoreType.DMA,
            pltpu.SemaphoreType.DMA,
            pltpu.SemaphoreType.DMA,
            pltpu.SemaphoreType.DMA,
            pltpu.SemaphoreType.DMA,
        ],
        compiler_params=pltpu.CompilerParams(use_tc_tiling_on_sc=False,
                                             needs_layout_passes=False),
    )
    def emb(idx_hbm, tok_hbm, pos_hbm, out_hbm,
            idxb_v, pos_v, gidx_v, g_v, s_v, pb_v, shared,
            sem_g0, sem_g1, sem_w0, sem_w1, sem_p, sem_x):
        sem_g = (sem_g0, sem_g1)
        sem_w = (sem_w0, sem_w1)
        cid = lax.axis_index("c")
        sid = lax.axis_index("s")
        wid = sid * 2 + cid

        pltpu.async_copy(pos_hbm.at[pl.ds(0, t)], pos_v, sem_x)
        pltpu.sync_copy(idx_hbm.at[pl.ds(wid * rpw, rpw)], idxb_v)
        pltpu.make_async_copy(pos_hbm.at[pl.ds(0, t)], pos_v, sem_x).wait()

        iota = jnp.arange(L, dtype=jnp.int32)
        iota_t = iota * t

        @pl.loop(sid, t, step=16)
        def build_pos(t2):
            tv = jnp.full((L,), t2, jnp.int32)
            for f in range(d_half):
                pv = plsc.load_gather(
                    pos_v, [tv, jnp.full((L,), f, jnp.int32)])
                for j in range(8):
                    pb_v[pl.ds(f * 128 + j * L, L)] = pv
            pltpu.sync_copy(pb_v, shared.at[t2])

        plsc.subcore_barrier()

        def extract_gidx(tt, p):
            for j in range(8):
                addr = iota_t + (j * L * t + tt)
                gidx_v[p, pl.ds(j * L, L)] = plsc.load_gather(idxb_v, [addr])

        def fire_gather(tt, p):
            extract_gidx(tt, p)
            pltpu.async_copy(tok_hbm.at[gidx_v.at[p]], g_v.at[p], sem_g[p])

        def sub(tt, p):
            @pl.when(tt >= 2)
            def _():
                for ft in range(4):
                    pltpu.make_async_copy(
                        s_v.at[p, pl.ds(ft * 1024, 1024)],
                        out_hbm.at[tt, pl.ds(ft * 32768 + wid * 1024, 1024)],
                        sem_w[p]).wait()

            pltpu.make_async_copy(
                tok_hbm.at[gidx_v.at[p]], g_v.at[p], sem_g[p]).wait()

            @pl.when(tt + 1 < t)
            def _():
                fire_gather(tt + 1, 1 - p)

            for ft in range(4):
                pltpu.async_copy(
                    shared.at[tt, pl.ds(ft * 1024, 1024)],
                    out_hbm.at[tt, pl.ds((4 + ft) * 32768 + wid * 1024,
                                         1024)],
                    sem_p)

            @plsc.parallel_loop(0, d_half)
            def transpose_f(f):
                col = jnp.full((L,), f, jnp.int32)
                for j in range(8):
                    v = plsc.load_gather(g_v.at[p], [iota + (j * L), col])
                    s_v[p, pl.ds(f * 128 + j * L, L)] = v

            for ft in range(4):
                pltpu.async_copy(
                    s_v.at[p, pl.ds(ft * 1024, 1024)],
                    out_hbm.at[tt, pl.ds(ft * 32768 + wid * 1024, 1024)],
                    sem_w[p])

        fire_gather(0, 0)

        @pl.loop(0, t, step=2)
        def body(g):
            sub(g, 0)
            sub(g + 1, 1)

        for p in range(2):
            for ft in range(4):
                pltpu.make_async_copy(
                    s_v.at[p, pl.ds(ft * 1024, 1024)],
                    out_hbm.at[0, pl.ds(ft * 32768 + wid * 1024, 1024)],
                    sem_w[p]).wait()

        @pl.loop(0, 4 * t)
        def drain_pos(i):
            pltpu.make_async_copy(
                shared.at[0, pl.ds(0, 1024)],
                out_hbm.at[0, pl.ds(wid * 1024, 1024)], sem_p).wait()

    return emb(idx_flat, tok_table, pos_table)


def kernel(idx, tok_table, pos_table):
    B, T = idx.shape
    d_half = tok_table.shape[1]
    out2 = _embed(B, T, d_half, idx.reshape(-1).astype(jnp.int32),
                  tok_table, pos_table)
    out5 = out2.reshape(T, 8, B // 128, 8, 128)
    return out5.transpose(2, 4, 0, 1, 3).reshape(B, T, 2 * d_half)


# speedup vs baseline: 1.4947x; 1.0036x over previous
import functools

import jax
import jax.numpy as jnp
from jax import lax
from jax.experimental import pallas as pl
from jax.experimental.pallas import tpu as pltpu
from jax.experimental.pallas import tpu_sc as plsc


@functools.partial(jax.jit, static_argnums=(0, 1, 2))
def _embed(nb, t, d_half, idx_flat, tok_table, pos_table):
    NW = 32
    L = 16
    assert nb == NW * 128
    assert d_half == 32
    rpw = 128 * t

    mesh = plsc.VectorSubcoreMesh(core_axis_name="c", subcore_axis_name="s")

    @functools.partial(
        pl.kernel,
        out_type=jax.ShapeDtypeStruct((t, 8 * NW * 8 * 128), jnp.float32),
        mesh=mesh,
        scratch_types=[
            pltpu.VMEM((rpw,), jnp.int32),
            pltpu.VMEM((t, d_half), jnp.float32),
            pltpu.VMEM((2, 128), jnp.int32),
            pltpu.VMEM((2, 128, 32), jnp.float32),
            pltpu.VMEM((2, 4 * 8 * 128), jnp.float32),
            pltpu.VMEM((4 * 8 * 128,), jnp.float32),
            pltpu.VMEM_SHARED((t, 4 * 8 * 128), jnp.float32),
            pltpu.SemaphoreType.DMA,
            pltpu.SemaphoreType.DMA,
            pltpu.SemaphoreType.DMA,
            pltpu.SemaphoreType.DMA,
            pltpu.SemaphoreType.DMA,
            pltpu.SemaphoreType.DMA,
        ],
        compiler_params=pltpu.CompilerParams(use_tc_tiling_on_sc=False,
                                             needs_layout_passes=False),
    )
    def emb(idx_hbm, tok_hbm, pos_hbm, out_hbm,
            idxb_v, pos_v, gidx_v, g_v, s_v, pb_v, shared,
            sem_g0, sem_g1, sem_w0, sem_w1, sem_p, sem_x):
        sem_g = (sem_g0, sem_g1)
        sem_w = (sem_w0, sem_w1)
        cid = lax.axis_index("c")
        sid = lax.axis_index("s")
        wid = sid * 2 + cid

        pltpu.async_copy(pos_hbm.at[pl.ds(0, t)], pos_v, sem_x)
        pltpu.sync_copy(idx_hbm.at[pl.ds(wid * rpw, rpw)], idxb_v)
        pltpu.make_async_copy(pos_hbm.at[pl.ds(0, t)], pos_v, sem_x).wait()

        iota = jnp.arange(L, dtype=jnp.int32)
        iota_t = iota * t

        @pl.loop(sid, t, step=16)
        def build_pos(t2):
            tv = jnp.full((L,), t2, jnp.int32)
            for f in range(d_half):
                pv = plsc.load_gather(
                    pos_v, [tv, jnp.full((L,), f, jnp.int32)])
                for j in range(8):
                    pb_v[pl.ds(f * 128 + j * L, L)] = pv
            pltpu.sync_copy(pb_v, shared.at[t2])

        plsc.subcore_barrier()

        def extract_gidx(tt, p):
            for j in range(8):
                addr = iota_t + (j * L * t + tt)
                gidx_v[p, pl.ds(j * L, L)] = plsc.load_gather(idxb_v, [addr])

        def fire_gather(tt, p):
            extract_gidx(tt, p)
            pltpu.async_copy(tok_hbm.at[gidx_v.at[p]], g_v.at[p], sem_g[p])

        def sub(tt, p):
            @pl.when(tt >= 2)
            def _():
                for ft in range(4):
                    pltpu.make_async_copy(
                        s_v.at[p, pl.ds(ft * 1024, 1024)],
                        out_hbm.at[tt, pl.ds(ft * 32768 + wid * 1024, 1024)],
                        sem_w[p]).wait()

            pltpu.make_async_copy(
                tok_hbm.at[gidx_v.at[p]], g_v.at[p], sem_g[p]).wait()

            @pl.when(tt + 1 < t)
            def _():
                fire_gather(tt + 1, 1 - p)

            for ft in range(4):
                pltpu.async_copy(
                    shared.at[tt, pl.ds(ft * 1024, 1024)],
                    out_hbm.at[tt, pl.ds((4 + ft) * 32768 + wid * 1024,
                                         1024)],
                    sem_p)

            @plsc.parallel_loop(0, d_half, unroll=4)
            def transpose_f(f):
                col = jnp.full((L,), f, jnp.int32)
                for j in range(8):
                    v = plsc.load_gather(g_v.at[p], [iota + (j * L), col])
                    s_v[p, pl.ds(f * 128 + j * L, L)] = v

            for ft in range(4):
                pltpu.async_copy(
                    s_v.at[p, pl.ds(ft * 1024, 1024)],
                    out_hbm.at[tt, pl.ds(ft * 32768 + wid * 1024, 1024)],
                    sem_w[p])

        fire_gather(0, 0)

        @pl.loop(0, t, step=2)
        def body(g):
            sub(g, 0)
            sub(g + 1, 1)

        for p in range(2):
            for ft in range(4):
                pltpu.make_async_copy(
                    s_v.at[p, pl.ds(ft * 1024, 1024)],
                    out_hbm.at[0, pl.ds(ft * 32768 + wid * 1024, 1024)],
                    sem_w[p]).wait()

        @pl.loop(0, 4 * t)
        def drain_pos(i):
            pltpu.make_async_copy(
                shared.at[0, pl.ds(0, 1024)],
                out_hbm.at[0, pl.ds(wid * 1024, 1024)], sem_p).wait()

    return emb(idx_flat, tok_table, pos_table)


def kernel(idx, tok_table, pos_table):
    B, T = idx.shape
    d_half = tok_table.shape[1]
    out2 = _embed(B, T, d_half, idx.reshape(-1).astype(jnp.int32),
                  tok_table, pos_table)
    out5 = out2.reshape(T, 8, B // 128, 8, 128)
    return out5.transpose(2, 4, 0, 1, 3).reshape(B, T, 2 * d_half)
